# baseline (speedup 1.0000x reference)
import functools

import jax
import jax.numpy as jnp
from jax import lax
from jax.experimental import pallas as pl
from jax.experimental.pallas import tpu as pltpu
from jax.experimental.pallas import tpu_sc as plsc

N = 8 * 224 * 224 * 96
NC, NS, L = 2, 16, 16
NW = NC * NS
PER_W = N // NW
CHUNK = 8192
ITERS = PER_W // CHUNK
GROUPS = CHUNK // L

_mesh = plsc.VectorSubcoreMesh(core_axis_name="c", subcore_axis_name="s")


@functools.partial(
    pl.kernel,
    mesh=_mesh,
    out_type=(
        jax.ShapeDtypeStruct((N,), jnp.float32),
        jax.ShapeDtypeStruct((N,), jnp.float32),
    ),
    scratch_types=[
        pltpu.VMEM((CHUNK,), jnp.float32),
        pltpu.VMEM((CHUNK,), jnp.float32),
        pltpu.VMEM((CHUNK,), jnp.float32),
        pltpu.VMEM((CHUNK,), jnp.float32),
    ],
)
def _exchange(x1_hbm, x2_hbm, o1_hbm, o2_hbm, a_v, b_v, c_v, d_v):
    wid = lax.axis_index("s") * NC + lax.axis_index("c")
    base = wid * PER_W
    even = (lax.iota(jnp.int32, L) % 2) == 0

    def outer(i, carry):
        off = base + i * CHUNK
        pltpu.sync_copy(x1_hbm.at[pl.ds(off, CHUNK)], a_v)
        pltpu.sync_copy(x2_hbm.at[pl.ds(off, CHUNK)], b_v)

        def inner(j, c):
            s = pl.ds(j * L, L)
            v1 = a_v[s]
            v2 = b_v[s]
            c_v[s] = jnp.where(even, v2, v1)
            d_v[s] = jnp.where(even, v1, v2)
            return c

        lax.fori_loop(0, GROUPS, inner, 0)
        pltpu.sync_copy(c_v, o1_hbm.at[pl.ds(off, CHUNK)])
        pltpu.sync_copy(d_v, o2_hbm.at[pl.ds(off, CHUNK)])
        return carry

    lax.fori_loop(0, ITERS, outer, 0)


def kernel(x1, x2):
    B, H, W, C = x1.shape
    o1, o2 = _exchange(x1.reshape(-1), x2.reshape(-1))
    return o1.reshape(B, C, H, W), o2.reshape(B, C, H, W)


# speedup vs baseline: 1.1797x; 1.1797x over previous
import functools

import jax
import jax.numpy as jnp
from jax import lax
from jax.experimental import pallas as pl
from jax.experimental.pallas import tpu as pltpu
from jax.experimental.pallas import tpu_sc as plsc

N = 8 * 224 * 224 * 96
NC, NS, L = 2, 16, 16
NW = NC * NS
PER_W = N // NW
CHUNK = 14336
ITERS = PER_W // CHUNK
HALF = ITERS // 2
GROUPS = CHUNK // L

_mesh = plsc.VectorSubcoreMesh(core_axis_name="c", subcore_axis_name="s")


@functools.partial(
    pl.kernel,
    mesh=_mesh,
    out_type=(
        jax.ShapeDtypeStruct((N,), jnp.float32),
        jax.ShapeDtypeStruct((N,), jnp.float32),
    ),
    scratch_types=[
        pltpu.VMEM((2, CHUNK), jnp.float32),
        pltpu.VMEM((2, CHUNK), jnp.float32),
        pltpu.VMEM((2, CHUNK), jnp.float32),
        pltpu.VMEM((2, CHUNK), jnp.float32),
        pltpu.SemaphoreType.DMA,
        pltpu.SemaphoreType.DMA,
        pltpu.SemaphoreType.DMA,
        pltpu.SemaphoreType.DMA,
    ],
)
def _exchange(x1_hbm, x2_hbm, o1_hbm, o2_hbm, a_v, b_v, c_v, d_v,
              ls0, ls1, ss0, ss1):
    wid = lax.axis_index("s") * NC + lax.axis_index("c")
    base = wid * PER_W
    even = (lax.iota(jnp.int32, L) % 2) == 0
    lsems = (ls0, ls1)
    ssems = (ss0, ss1)

    def start_load(slot, i):
        off = base + i * CHUNK
        pltpu.async_copy(x1_hbm.at[pl.ds(off, CHUNK)], a_v.at[slot],
                         lsems[slot])
        pltpu.async_copy(x2_hbm.at[pl.ds(off, CHUNK)], b_v.at[slot],
                         lsems[slot])

    def wait_load(slot):
        pltpu.make_async_copy(x1_hbm.at[pl.ds(0, CHUNK)], a_v.at[slot],
                              lsems[slot]).wait()
        pltpu.make_async_copy(x2_hbm.at[pl.ds(0, CHUNK)], b_v.at[slot],
                              lsems[slot]).wait()

    def start_store(slot, i):
        off = base + i * CHUNK
        pltpu.async_copy(c_v.at[slot], o1_hbm.at[pl.ds(off, CHUNK)],
                         ssems[slot])
        pltpu.async_copy(d_v.at[slot], o2_hbm.at[pl.ds(off, CHUNK)],
                         ssems[slot])

    def wait_store(slot):
        pltpu.make_async_copy(c_v.at[slot], o1_hbm.at[pl.ds(0, CHUNK)],
                              ssems[slot]).wait()
        pltpu.make_async_copy(d_v.at[slot], o2_hbm.at[pl.ds(0, CHUNK)],
                              ssems[slot]).wait()

    start_load(0, 0)
    start_load(1, 1)

    def body(k, carry):
        for slot in (0, 1):
            i = 2 * k + slot
            wait_load(slot)
            pl.when(k > 0)(lambda: wait_store(slot))

            ain = a_v.at[slot]
            bin_ = b_v.at[slot]
            cout = c_v.at[slot]
            dout = d_v.at[slot]

            @plsc.parallel_loop(0, GROUPS, 1, unroll=8)
            def _(j):
                s = pl.ds(j * L, L)
                v1 = ain[s]
                v2 = bin_[s]
                cout[s] = jnp.where(even, v2, v1)
                dout[s] = jnp.where(even, v1, v2)

            start_store(slot, i)
            pl.when(k < HALF - 1)(lambda: start_load(slot, i + 2))
        return carry

    lax.fori_loop(0, HALF, body, 0)
    wait_store(0)
    wait_store(1)


def kernel(x1, x2):
    B, H, W, C = x1.shape
    o1, o2 = _exchange(x1.reshape(-1), x2.reshape(-1))
    return o1.reshape(B, C, H, W), o2.reshape(B, C, H, W)


# speedup vs baseline: 1.2168x; 1.0314x over previous
import functools

import jax
import jax.numpy as jnp
from jax import lax
from jax.experimental import pallas as pl
from jax.experimental.pallas import tpu as pltpu
from jax.experimental.pallas import tpu_sc as plsc

N = 8 * 224 * 224 * 96
R = N // 128
NC, NS, L = 2, 16, 16
NW = NC * NS
PER_W = R // NW
CR = 112
ITERS = PER_W // CR
HALF = ITERS // 2
GROUPS = 128 // L

_mesh = plsc.VectorSubcoreMesh(core_axis_name="c", subcore_axis_name="s")


@functools.partial(
    pl.kernel,
    mesh=_mesh,
    out_type=(
        jax.ShapeDtypeStruct((R, 128), jnp.float32),
        jax.ShapeDtypeStruct((R, 128), jnp.float32),
    ),
    scratch_types=[
        pltpu.VMEM((2, CR, 128), jnp.float32),
        pltpu.VMEM((2, CR, 128), jnp.float32),
        pltpu.VMEM((2, CR, 128), jnp.float32),
        pltpu.VMEM((2, CR, 128), jnp.float32),
        pltpu.SemaphoreType.DMA,
        pltpu.SemaphoreType.DMA,
        pltpu.SemaphoreType.DMA,
        pltpu.SemaphoreType.DMA,
    ],
)
def _exchange(x1_hbm, x2_hbm, o1_hbm, o2_hbm, a_v, b_v, c_v, d_v,
              ls0, ls1, ss0, ss1):
    wid = lax.axis_index("s") * NC + lax.axis_index("c")
    base = wid * PER_W
    even = (lax.iota(jnp.int32, L) % 2) == 0
    lsems = (ls0, ls1)
    ssems = (ss0, ss1)

    def start_load(slot, i):
        off = base + i * CR
        pltpu.async_copy(x1_hbm.at[pl.ds(off, CR)], a_v.at[slot],
                         lsems[slot])
        pltpu.async_copy(x2_hbm.at[pl.ds(off, CR)], b_v.at[slot],
                         lsems[slot])

    def wait_load(slot):
        pltpu.make_async_copy(x1_hbm.at[pl.ds(0, CR)], a_v.at[slot],
                              lsems[slot]).wait()
        pltpu.make_async_copy(x2_hbm.at[pl.ds(0, CR)], b_v.at[slot],
                              lsems[slot]).wait()

    def start_store(slot, i):
        off = base + i * CR
        pltpu.async_copy(c_v.at[slot], o1_hbm.at[pl.ds(off, CR)],
                         ssems[slot])
        pltpu.async_copy(d_v.at[slot], o2_hbm.at[pl.ds(off, CR)],
                         ssems[slot])

    def wait_store(slot):
        pltpu.make_async_copy(c_v.at[slot], o1_hbm.at[pl.ds(0, CR)],
                              ssems[slot]).wait()
        pltpu.make_async_copy(d_v.at[slot], o2_hbm.at[pl.ds(0, CR)],
                              ssems[slot]).wait()

    start_load(0, 0)
    start_load(1, 1)

    def body(k, carry):
        for slot in (0, 1):
            i = 2 * k + slot
            wait_load(slot)
            pl.when(k > 0)(lambda: wait_store(slot))

            ain = a_v.at[slot]
            bin_ = b_v.at[slot]
            cout = c_v.at[slot]
            dout = d_v.at[slot]

            @plsc.parallel_loop(0, CR, 1, unroll=2)
            def _(r):
                for g in range(GROUPS):
                    s = pl.ds(g * L, L)
                    v1 = ain[r, s]
                    v2 = bin_[r, s]
                    cout[r, s] = jnp.where(even, v2, v1)
                    dout[r, s] = jnp.where(even, v1, v2)

            start_store(slot, i)
            pl.when(k < HALF - 1)(lambda: start_load(slot, i + 2))
        return carry

    lax.fori_loop(0, HALF, body, 0)
    wait_store(0)
    wait_store(1)


def kernel(x1, x2):
    B, H, W, C = x1.shape
    o1, o2 = _exchange(x1.reshape(R, 128), x2.reshape(R, 128))
    return o1.reshape(B, C, H, W), o2.reshape(B, C, H, W)


# speedup vs baseline: 1.2174x; 1.0005x over previous
import functools

import jax
import jax.numpy as jnp
from jax import lax
from jax.experimental import pallas as pl
from jax.experimental.pallas import tpu as pltpu
from jax.experimental.pallas import tpu_sc as plsc

N = 8 * 224 * 224 * 96
R = N // 128
NC, NS, L = 2, 16, 16
NW = NC * NS
PER_W = R // NW
CR = 112
ITERS = PER_W // CR
HALF = ITERS // 2
GROUPS = 128 // L

_mesh = plsc.VectorSubcoreMesh(core_axis_name="c", subcore_axis_name="s")


@functools.partial(
    pl.kernel,
    mesh=_mesh,
    out_type=(
        jax.ShapeDtypeStruct((R, 128), jnp.float32),
        jax.ShapeDtypeStruct((R, 128), jnp.float32),
    ),
    scratch_types=[
        pltpu.VMEM((2, CR, 128), jnp.float32),
        pltpu.VMEM((2, CR, 128), jnp.float32),
        pltpu.VMEM((2, CR, 128), jnp.float32),
        pltpu.VMEM((2, CR, 128), jnp.float32),
        pltpu.SemaphoreType.DMA,
        pltpu.SemaphoreType.DMA,
        pltpu.SemaphoreType.DMA,
        pltpu.SemaphoreType.DMA,
    ],
    compiler_params=pltpu.CompilerParams(use_tc_tiling_on_sc=True),
)
def _exchange(x1_hbm, x2_hbm, o1_hbm, o2_hbm, a_v, b_v, c_v, d_v,
              ls0, ls1, ss0, ss1):
    wid = lax.axis_index("s") * NC + lax.axis_index("c")
    base = wid * PER_W
    even = (lax.iota(jnp.int32, L) % 2) == 0
    lsems = (ls0, ls1)
    ssems = (ss0, ss1)

    def start_load(slot, i):
        off = base + i * CR
        pltpu.async_copy(x1_hbm.at[pl.ds(off, CR)], a_v.at[slot],
                         lsems[slot])
        pltpu.async_copy(x2_hbm.at[pl.ds(off, CR)], b_v.at[slot],
                         lsems[slot])

    def wait_load(slot):
        pltpu.make_async_copy(x1_hbm.at[pl.ds(0, CR)], a_v.at[slot],
                              lsems[slot]).wait()
        pltpu.make_async_copy(x2_hbm.at[pl.ds(0, CR)], b_v.at[slot],
                              lsems[slot]).wait()

    def start_store(slot, i):
        off = base + i * CR
        pltpu.async_copy(c_v.at[slot], o1_hbm.at[pl.ds(off, CR)],
                         ssems[slot])
        pltpu.async_copy(d_v.at[slot], o2_hbm.at[pl.ds(off, CR)],
                         ssems[slot])

    def wait_store(slot):
        pltpu.make_async_copy(c_v.at[slot], o1_hbm.at[pl.ds(0, CR)],
                              ssems[slot]).wait()
        pltpu.make_async_copy(d_v.at[slot], o2_hbm.at[pl.ds(0, CR)],
                              ssems[slot]).wait()

    start_load(0, 0)
    start_load(1, 1)

    def body(k, carry):
        for slot in (0, 1):
            i = 2 * k + slot
            wait_load(slot)
            pl.when(k > 0)(lambda: wait_store(slot))

            ain = a_v.at[slot]
            bin_ = b_v.at[slot]
            cout = c_v.at[slot]
            dout = d_v.at[slot]

            @plsc.parallel_loop(0, CR, 1, unroll=2)
            def _(r):
                for g in range(GROUPS):
                    s = pl.ds(g * L, L)
                    v1 = ain[r, s]
                    v2 = bin_[r, s]
                    cout[r, s] = jnp.where(even, v2, v1)
                    dout[r, s] = jnp.where(even, v1, v2)

            start_store(slot, i)
            pl.when(k < HALF - 1)(lambda: start_load(slot, i + 2))
        return carry

    lax.fori_loop(0, HALF, body, 0)
    wait_store(0)
    wait_store(1)


def kernel(x1, x2):
    B, H, W, C = x1.shape
    o1, o2 = _exchange(x1.reshape(R, 128), x2.reshape(R, 128))
    return o1.reshape(B, C, H, W), o2.reshape(B, C, H, W)


# speedup vs baseline: 1.3322x; 1.0943x over previous
import functools

import jax
import jax.numpy as jnp
from jax import lax
from jax.experimental import pallas as pl
from jax.experimental.pallas import tpu as pltpu
from jax.experimental.pallas import tpu_sc as plsc

B, H, W, C = 8, 224, 224, 96
HW = H * W
N = B * HW * C
R = N // 128
NC, NS, L = 2, 16, 16
NW = NC * NS
PER_W = (B * HW) // NW
CRW = 128
ITERS = PER_W // CRW
HALF = ITERS // 2
CG = C // L
OROWS = CRW * C // 128

_mesh = plsc.VectorSubcoreMesh(core_axis_name="c", subcore_axis_name="s")


@functools.partial(
    pl.kernel,
    mesh=_mesh,
    out_type=(
        jax.ShapeDtypeStruct((R, 128), jnp.float32),
        jax.ShapeDtypeStruct((R, 128), jnp.float32),
    ),
    scratch_types=[
        pltpu.VMEM((2, CRW, C), jnp.float32),
        pltpu.VMEM((2, CRW, C), jnp.float32),
        pltpu.VMEM((2, OROWS, 128), jnp.float32),
        pltpu.VMEM((2, OROWS, 128), jnp.float32),
        pltpu.SemaphoreType.DMA,
        pltpu.SemaphoreType.DMA,
        pltpu.SemaphoreType.DMA,
        pltpu.SemaphoreType.DMA,
    ],
)
def _exchange(x1_hbm, x2_hbm, o1_hbm, o2_hbm, a_v, b_v, c_v, d_v,
              ls0, ls1, ss0, ss1):
    wid = lax.axis_index("s") * NC + lax.axis_index("c")
    bidx = wid // 4
    rec0 = (wid % 4) * PER_W
    even = (lax.iota(jnp.int32, L) % 2) == 0
    lsems = (ls0, ls1)
    ssems = (ss0, ss1)

    def start_load(slot, i):
        off = pl.multiple_of(rec0 + i * CRW, 8)
        pltpu.async_copy(x1_hbm.at[bidx, pl.ds(off, CRW)], a_v.at[slot],
                         lsems[slot])
        pltpu.async_copy(x2_hbm.at[bidx, pl.ds(off, CRW)], b_v.at[slot],
                         lsems[slot])

    def wait_load(slot):
        pltpu.make_async_copy(x1_hbm.at[0, pl.ds(0, CRW)], a_v.at[slot],
                              lsems[slot]).wait()
        pltpu.make_async_copy(x2_hbm.at[0, pl.ds(0, CRW)], b_v.at[slot],
                              lsems[slot]).wait()

    def start_store(slot, i):
        orow = pl.multiple_of((bidx * HW + rec0 + i * CRW) * C // 128, 8)
        pltpu.async_copy(c_v.at[slot], o1_hbm.at[pl.ds(orow, OROWS)],
                         ssems[slot])
        pltpu.async_copy(d_v.at[slot], o2_hbm.at[pl.ds(orow, OROWS)],
                         ssems[slot])

    def wait_store(slot):
        pltpu.make_async_copy(c_v.at[slot], o1_hbm.at[pl.ds(0, OROWS)],
                              ssems[slot]).wait()
        pltpu.make_async_copy(d_v.at[slot], o2_hbm.at[pl.ds(0, OROWS)],
                              ssems[slot]).wait()

    start_load(0, 0)
    start_load(1, 1)

    def body(k, carry):
        for slot in (0, 1):
            i = 2 * k + slot
            wait_load(slot)
            pl.when(k > 0)(lambda: wait_store(slot))

            ain = a_v.at[slot]
            bin_ = b_v.at[slot]
            cout = c_v.at[slot]
            dout = d_v.at[slot]

            @plsc.parallel_loop(0, CRW, 1, unroll=2)
            def _(r):
                for g in range(CG):
                    v1 = ain[r, pl.ds(g * L, L)]
                    v2 = bin_[r, pl.ds(g * L, L)]
                    p = r * C + g * L
                    row = p // 128
                    lane = p % 128
                    cout[row, pl.ds(lane, L)] = jnp.where(even, v2, v1)
                    dout[row, pl.ds(lane, L)] = jnp.where(even, v1, v2)

            start_store(slot, i)
            pl.when(k < HALF - 1)(lambda: start_load(slot, i + 2))
        return carry

    lax.fori_loop(0, HALF, body, 0)
    wait_store(0)
    wait_store(1)


def kernel(x1, x2):
    o1, o2 = _exchange(x1.reshape(B, HW, C), x2.reshape(B, HW, C))
    return o1.reshape(B, C, H, W), o2.reshape(B, C, H, W)
